# Initial kernel scaffold; baseline (speedup 1.0000x reference)
import jax
import jax.numpy as jnp
from jax.experimental import pallas as pl


def kernel(pos, atomic_numbers, edge_index, distance_vec, batch, atom_emb, centers, W_msg, W_rbf, W_upd, W_energy, W_force, W1, b1, W2, b2):
    raise NotImplementedError("write your pallas kernel here")


# speedup vs baseline: 1.1075x; 1.1075x over previous
import functools
import jax
import jax.numpy as jnp
from jax.experimental import pallas as pl
from jax.experimental.pallas import tpu as pltpu

BN = 1000
BE = 1600


def _prep_body(an_ref, emb_ref, wmsg_ref, h_ref, g_ref):
    an = an_ref[0]
    nelem = emb_ref.shape[0]
    oh = (jax.lax.broadcasted_iota(jnp.int32, (BN, nelem), 1) == an
          ).astype(jnp.float32)
    h = jnp.dot(oh, emb_ref[...], preferred_element_type=jnp.float32)
    h_ref[...] = h
    g_ref[...] = jnp.dot(h, wmsg_ref[...], preferred_element_type=jnp.float32)


def _edge_body(dv_ref, gs_ref, gd_ref, cen_ref, wrbf_ref, wf_ref, out_ref):
    dv = dv_ref[...]
    d2 = jnp.sum(dv * dv, axis=1, keepdims=True)
    d = jnp.sqrt(d2 + 1e-12)
    rbf = jnp.exp(-((d - cen_ref[...]) ** 2))
    t = jnp.dot(rbf, wrbf_ref[...], preferred_element_type=jnp.float32)
    a = gs_ref[...] + gd_ref[...] + t
    m = a * jax.lax.logistic(a)
    fscal = jnp.dot(m, wf_ref[...], preferred_element_type=jnp.float32)
    fe = fscal * (dv / d)
    out_ref[...] = jnp.concatenate(
        [m, fe, jnp.zeros((BE, 13), jnp.float32)], axis=1)


def _final_body(h_ref, p_ref, batch_ref, wupd_ref, we_ref, b1_ref, w2_ref,
                b2_ref, f_ref, e_ref, s_ref, cnt_ref):
    i = pl.program_id(0)
    nsteps = pl.num_programs(0)
    agg = p_ref[0, :, :128] + p_ref[1, :, :128]
    fsum = p_ref[0, :, 128:131] + p_ref[1, :, 128:131]
    f_ref[...] = jnp.concatenate(
        [fsum, jnp.zeros((BN, 13), jnp.float32)], axis=1)
    hn = h_ref[...] + jnp.dot(agg, wupd_ref[...],
                              preferred_element_type=jnp.float32)
    e = jnp.dot(hn, we_ref[...], preferred_element_type=jnp.float32)
    batch_row = batch_ref[0]
    s_dim = e_ref.shape[0]
    oht = (jax.lax.broadcasted_iota(jnp.int32, (s_dim, BN), 0) == batch_row
           ).astype(jnp.float32)
    e_part = jnp.dot(oht, e, preferred_element_type=jnp.float32)
    c_part = jnp.sum(oht, axis=1, keepdims=True)

    @pl.when(i == 0)
    def _():
        e_ref[...] = jnp.zeros_like(e_ref)
        cnt_ref[...] = jnp.zeros_like(cnt_ref)

    e_ref[...] += e_part
    cnt_ref[...] += c_part

    @pl.when(i == nsteps - 1)
    def _():
        srow = jnp.dot(jnp.tanh(b1_ref[...]), w2_ref[...],
                       preferred_element_type=jnp.float32) + b2_ref[...]
        s_ref[...] = cnt_ref[...] * srow


def _prep(atomic_numbers, atom_emb, W_msg):
    n = atomic_numbers.shape[0]
    nelem, d = atom_emb.shape
    an3 = atomic_numbers.reshape(n // BN, BN, 1)
    return pl.pallas_call(
        _prep_body,
        grid=(n // BN,),
        in_specs=[
            pl.BlockSpec((1, BN, 1), lambda i: (i, 0, 0)),
            pl.BlockSpec((nelem, d), lambda i: (0, 0)),
            pl.BlockSpec((d, d), lambda i: (0, 0)),
        ],
        out_specs=[
            pl.BlockSpec((BN, d), lambda i: (i, 0)),
            pl.BlockSpec((BN, d), lambda i: (i, 0)),
        ],
        out_shape=[
            jax.ShapeDtypeStruct((n, d), jnp.float32),
            jax.ShapeDtypeStruct((n, d), jnp.float32),
        ],
    )(an3, atom_emb, W_msg)


def _edge(distance_vec, g_src, g_dst, centers, W_rbf, W_force):
    e, d = g_src.shape
    nrbf = W_rbf.shape[0]
    cen2 = centers.reshape(1, nrbf)
    return pl.pallas_call(
        _edge_body,
        grid=(e // BE,),
        in_specs=[
            pl.BlockSpec((BE, 3), lambda i: (i, 0)),
            pl.BlockSpec((BE, d), lambda i: (i, 0)),
            pl.BlockSpec((BE, d), lambda i: (i, 0)),
            pl.BlockSpec((1, nrbf), lambda i: (0, 0)),
            pl.BlockSpec((nrbf, d), lambda i: (0, 0)),
            pl.BlockSpec((d, 1), lambda i: (0, 0)),
        ],
        out_specs=pl.BlockSpec((BE, 144), lambda i: (i, 0)),
        out_shape=jax.ShapeDtypeStruct((e, 144), jnp.float32),
    )(distance_vec, g_src, g_dst, cen2, W_rbf, W_force)


def _final(h, partials, batch, W_upd, W_energy, b1, W2, b2, num_struct):
    n, d = h.shape
    batch3 = batch.reshape(n // BN, 1, BN)
    return pl.pallas_call(
        _final_body,
        grid=(n // BN,),
        in_specs=[
            pl.BlockSpec((BN, d), lambda i: (i, 0)),
            pl.BlockSpec((2, BN, 144), lambda i: (0, i, 0)),
            pl.BlockSpec((1, 1, BN), lambda i: (i, 0, 0)),
            pl.BlockSpec((d, d), lambda i: (0, 0)),
            pl.BlockSpec((d, 1), lambda i: (0, 0)),
            pl.BlockSpec((1, d), lambda i: (0, 0)),
            pl.BlockSpec((d, 6), lambda i: (0, 0)),
            pl.BlockSpec((1, 6), lambda i: (0, 0)),
        ],
        out_specs=[
            pl.BlockSpec((BN, 16), lambda i: (i, 0)),
            pl.BlockSpec((num_struct, 1), lambda i: (0, 0)),
            pl.BlockSpec((num_struct, 6), lambda i: (0, 0)),
        ],
        out_shape=[
            jax.ShapeDtypeStruct((n, 16), jnp.float32),
            jax.ShapeDtypeStruct((num_struct, 1), jnp.float32),
            jax.ShapeDtypeStruct((num_struct, 6), jnp.float32),
        ],
        scratch_shapes=[pltpu.VMEM((num_struct, 1), jnp.float32)],
    )(h, partials, batch3, W_upd, W_energy, b1.reshape(1, d), W2,
      b2.reshape(1, 6))


def kernel(pos, atomic_numbers, edge_index, distance_vec, batch, atom_emb,
           centers, W_msg, W_rbf, W_upd, W_energy, W_force, W1, b1, W2, b2):
    n = pos.shape[0]
    src = edge_index[0]
    dst = edge_index[1]
    h, g = _prep(atomic_numbers, atom_emb, W_msg)
    g_src = jnp.take(g, src, axis=0)
    g_dst = jnp.take(g, dst, axis=0)
    edge_out = _edge(distance_vec, g_src, g_dst, centers, W_rbf, W_force)
    p = jax.ops.segment_sum(edge_out, dst, num_segments=n)
    partials = jnp.stack([p, jnp.zeros_like(p)])
    f16, e_out, stress = _final(h, partials, batch, W_upd, W_energy, b1, W2,
                                b2, 64)
    forces = f16[:, :3]
    energy = e_out[:, 0]
    return forces, energy, stress


# speedup vs baseline: 2.9588x; 2.6716x over previous
import functools
import jax
import jax.numpy as jnp
from jax import lax
from jax.experimental import pallas as pl
from jax.experimental.pallas import tpu as pltpu
from jax.experimental.pallas import tpu_sc as plsc

BN = 1000
BE = 1600
NC, NS = 2, 16
NW = NC * NS
CG = 80
CS = 80


def _sc_gather(g, src, dst):
    e = src.shape[0]
    n, d = g.shape
    per_w = e // NW
    mesh = plsc.VectorSubcoreMesh(core_axis_name="c", subcore_axis_name="s",
                                  num_cores=NC, num_subcores=NS)

    @functools.partial(
        pl.kernel,
        out_type=[jax.ShapeDtypeStruct((e, d), jnp.float32),
                  jax.ShapeDtypeStruct((e, d), jnp.float32)],
        mesh=mesh,
        scratch_types=[pltpu.VMEM((CG,), jnp.int32),
                       pltpu.VMEM((CG, d), jnp.float32),
                       pltpu.SemaphoreType.DMA],
    )
    def k(g_hbm, src_hbm, dst_hbm, gs_hbm, gd_hbm, idx_v, rows_v, sem):
        wid = lax.axis_index("s") * NC + lax.axis_index("c")
        w0 = wid * per_w

        @pl.loop(0, per_w, step=CG)
        def _(c):
            base = w0 + c
            pltpu.sync_copy(src_hbm.at[pl.ds(base, CG)], idx_v)
            pltpu.async_copy(g_hbm.at[idx_v], rows_v, sem).wait()
            pltpu.sync_copy(rows_v, gs_hbm.at[pl.ds(base, CG)])
            pltpu.sync_copy(dst_hbm.at[pl.ds(base, CG)], idx_v)
            pltpu.async_copy(g_hbm.at[idx_v], rows_v, sem).wait()
            pltpu.sync_copy(rows_v, gd_hbm.at[pl.ds(base, CG)])

    return k(g, src, dst)


def _sc_scatter16(edge16, dst, zeros16):
    e, d = edge16.shape
    n = zeros16.shape[0]
    per_core = e // NC
    per_sub = per_core // NS
    nz = n // 10
    mesh = plsc.VectorSubcoreMesh(core_axis_name="c", subcore_axis_name="s",
                                  num_cores=NC, num_subcores=NS)

    @functools.partial(
        pl.kernel,
        out_type=jax.ShapeDtypeStruct((NC * n, d), jnp.float32),
        mesh=mesh,
        scratch_types=[pltpu.VMEM_SHARED((n, d), jnp.float32),
                       pltpu.VMEM((CS,), jnp.int32),
                       pltpu.VMEM((CS, d), jnp.float32),
                       pltpu.SemaphoreType.DMA],
    )
    def k(eo_hbm, dst_hbm, z_hbm, out_hbm, acc, idx_v, buf, sem):
        cid = lax.axis_index("c")
        sid = lax.axis_index("s")

        @pl.when(sid < 10)
        def _():
            pltpu.sync_copy(z_hbm.at[pl.ds(sid * nz, nz)],
                            acc.at[pl.ds(sid * nz, nz)])
        plsc.subcore_barrier()

        base0 = cid * per_core + sid * per_sub

        @pl.loop(0, per_sub, step=CS)
        def _(c):
            base = base0 + c
            pltpu.sync_copy(dst_hbm.at[pl.ds(base, CS)], idx_v)
            pltpu.sync_copy(eo_hbm.at[pl.ds(base, CS)], buf)
            pltpu.sync_copy(buf, acc.at[idx_v], add=True)
        plsc.subcore_barrier()

        @pl.when(sid < 10)
        def _():
            pltpu.sync_copy(acc.at[pl.ds(sid * nz, nz)],
                            out_hbm.at[pl.ds(cid * n + sid * nz, nz)])

    return k(edge16, dst, zeros16)


def _prep_body(an_ref, emb_ref, wmsg_ref, we_ref, g_ref, he_ref):
    an = an_ref[0]
    nelem = emb_ref.shape[0]
    oh = (jax.lax.broadcasted_iota(jnp.int32, (BN, nelem), 1) == an
          ).astype(jnp.float32)
    h = jnp.dot(oh, emb_ref[...], preferred_element_type=jnp.float32)
    g_ref[...] = jnp.dot(h, wmsg_ref[...], preferred_element_type=jnp.float32)
    he_ref[...] = jnp.dot(h, we_ref[...], preferred_element_type=jnp.float32)


def _edge_body(dv_ref, gs_ref, gd_ref, cen_ref, wrbf_ref, wf_ref, wupd_ref,
               we_ref, out_ref):
    dv = dv_ref[...]
    d2 = jnp.sum(dv * dv, axis=1, keepdims=True)
    d = jnp.sqrt(d2 + 1e-12)
    rbf = jnp.exp(-((d - cen_ref[...]) ** 2))
    t = jnp.dot(rbf, wrbf_ref[...], preferred_element_type=jnp.float32)
    a = gs_ref[...] + gd_ref[...] + t
    m = a * jax.lax.logistic(a)
    w = jnp.dot(wupd_ref[...], we_ref[...], preferred_element_type=jnp.float32)
    q = jnp.dot(m, w, preferred_element_type=jnp.float32)
    fscal = jnp.dot(m, wf_ref[...], preferred_element_type=jnp.float32)
    fe = fscal * (dv / d)
    out_ref[...] = jnp.concatenate(
        [q, fe, jnp.zeros((BE, 124), jnp.float32)], axis=1)


def _final_body(he_ref, p_ref, batch_ref, b1_ref, w2_ref, b2_ref,
                f_ref, e_ref, s_ref, cnt_ref):
    i = pl.program_id(0)
    nsteps = pl.num_programs(0)
    psum = p_ref[0] + p_ref[1]
    f_ref[...] = jnp.concatenate(
        [psum[:, 1:4], jnp.zeros((BN, 13), jnp.float32)], axis=1)
    e_node = he_ref[...] + psum[:, 0:1]
    batch_row = batch_ref[0]
    s_dim = e_ref.shape[0]
    oht = (jax.lax.broadcasted_iota(jnp.int32, (s_dim, BN), 0) == batch_row
           ).astype(jnp.float32)
    e_part = jnp.dot(oht, e_node, preferred_element_type=jnp.float32)
    c_part = jnp.sum(oht, axis=1, keepdims=True)

    @pl.when(i == 0)
    def _():
        e_ref[...] = jnp.zeros_like(e_ref)
        cnt_ref[...] = jnp.zeros_like(cnt_ref)

    e_ref[...] += e_part
    cnt_ref[...] += c_part

    @pl.when(i == nsteps - 1)
    def _():
        srow = jnp.dot(jnp.tanh(b1_ref[...]), w2_ref[...],
                       preferred_element_type=jnp.float32) + b2_ref[...]
        s_ref[...] = cnt_ref[...] * srow


def _prep(atomic_numbers, atom_emb, W_msg, W_energy):
    n = atomic_numbers.shape[0]
    nelem, d = atom_emb.shape
    an3 = atomic_numbers.reshape(n // BN, BN, 1)
    return pl.pallas_call(
        _prep_body,
        grid=(n // BN,),
        in_specs=[
            pl.BlockSpec((1, BN, 1), lambda i: (i, 0, 0)),
            pl.BlockSpec((nelem, d), lambda i: (0, 0)),
            pl.BlockSpec((d, d), lambda i: (0, 0)),
            pl.BlockSpec((d, 1), lambda i: (0, 0)),
        ],
        out_specs=[
            pl.BlockSpec((BN, d), lambda i: (i, 0)),
            pl.BlockSpec((BN, 1), lambda i: (i, 0)),
        ],
        out_shape=[
            jax.ShapeDtypeStruct((n, d), jnp.float32),
            jax.ShapeDtypeStruct((n, 1), jnp.float32),
        ],
    )(an3, atom_emb, W_msg, W_energy)


def _edge(distance_vec, g_src, g_dst, centers, W_rbf, W_force, W_upd,
          W_energy):
    e, d = g_src.shape
    nrbf = W_rbf.shape[0]
    cen2 = centers.reshape(1, nrbf)
    return pl.pallas_call(
        _edge_body,
        grid=(e // BE,),
        in_specs=[
            pl.BlockSpec((BE, 3), lambda i: (i, 0)),
            pl.BlockSpec((BE, d), lambda i: (i, 0)),
            pl.BlockSpec((BE, d), lambda i: (i, 0)),
            pl.BlockSpec((1, nrbf), lambda i: (0, 0)),
            pl.BlockSpec((nrbf, d), lambda i: (0, 0)),
            pl.BlockSpec((d, 1), lambda i: (0, 0)),
            pl.BlockSpec((d, d), lambda i: (0, 0)),
            pl.BlockSpec((d, 1), lambda i: (0, 0)),
        ],
        out_specs=pl.BlockSpec((BE, 128), lambda i: (i, 0)),
        out_shape=jax.ShapeDtypeStruct((e, 128), jnp.float32),
    )(distance_vec, g_src, g_dst, cen2, W_rbf, W_force, W_upd, W_energy)


def _final(he, partials, batch, b1, W2, b2, num_struct):
    n = he.shape[0]
    d = b1.shape[0]
    batch3 = batch.reshape(n // BN, 1, BN)
    return pl.pallas_call(
        _final_body,
        grid=(n // BN,),
        in_specs=[
            pl.BlockSpec((BN, 1), lambda i: (i, 0)),
            pl.BlockSpec((2, BN, 128), lambda i: (0, i, 0)),
            pl.BlockSpec((1, 1, BN), lambda i: (i, 0, 0)),
            pl.BlockSpec((1, d), lambda i: (0, 0)),
            pl.BlockSpec((d, 6), lambda i: (0, 0)),
            pl.BlockSpec((1, 6), lambda i: (0, 0)),
        ],
        out_specs=[
            pl.BlockSpec((BN, 16), lambda i: (i, 0)),
            pl.BlockSpec((num_struct, 1), lambda i: (0, 0)),
            pl.BlockSpec((num_struct, 6), lambda i: (0, 0)),
        ],
        out_shape=[
            jax.ShapeDtypeStruct((n, 16), jnp.float32),
            jax.ShapeDtypeStruct((num_struct, 1), jnp.float32),
            jax.ShapeDtypeStruct((num_struct, 6), jnp.float32),
        ],
        scratch_shapes=[pltpu.VMEM((num_struct, 1), jnp.float32)],
    )(he, partials, batch3, b1.reshape(1, d), W2, b2.reshape(1, 6))


def kernel(pos, atomic_numbers, edge_index, distance_vec, batch, atom_emb,
           centers, W_msg, W_rbf, W_upd, W_energy, W_force, W1, b1, W2, b2):
    n = pos.shape[0]
    src = edge_index[0]
    dst = edge_index[1]
    g, he = _prep(atomic_numbers, atom_emb, W_msg, W_energy)
    g_src, g_dst = _sc_gather(g, src, dst)
    edge16 = _edge(distance_vec, g_src, g_dst, centers, W_rbf, W_force,
                   W_upd, W_energy)
    zeros_acc = jnp.zeros((n, 128), jnp.float32)
    p2 = _sc_scatter16(edge16, dst, zeros_acc)
    partials = p2.reshape(2, n, 128)
    f16, e_out, stress = _final(he, partials, batch, b1, W2, b2, 64)
    forces = f16[:, :3]
    energy = e_out[:, 0]
    return forces, energy, stress


# speedup vs baseline: 3.6323x; 1.2276x over previous
import functools
import jax
import jax.numpy as jnp
from jax import lax
from jax.experimental import pallas as pl
from jax.experimental.pallas import tpu as pltpu
from jax.experimental.pallas import tpu_sc as plsc

BN = 1000
BE = 1600
NC, NS = 2, 16
NW = NC * NS
CG = 80
CS = 80


def _sc_gather(g, src, dst):
    e = src.shape[0]
    n, d = g.shape
    per_w = e // NW
    nch = per_w // CG
    mesh = plsc.VectorSubcoreMesh(core_axis_name="c", subcore_axis_name="s",
                                  num_cores=NC, num_subcores=NS)
    src4 = src.reshape(NW, nch, 1, CG)
    dst4 = dst.reshape(NW, nch, 1, CG)

    @functools.partial(
        pl.kernel,
        out_type=[jax.ShapeDtypeStruct((e, d), jnp.float32),
                  jax.ShapeDtypeStruct((e, d), jnp.float32)],
        mesh=mesh,
        scratch_types=[pltpu.VMEM((nch, 1, CG), jnp.int32),
                       pltpu.VMEM((nch, 1, CG), jnp.int32),
                       pltpu.VMEM((CG, d), jnp.float32),
                       pltpu.VMEM((CG, d), jnp.float32),
                       pltpu.SemaphoreType.DMA,
                       pltpu.SemaphoreType.DMA,
                       pltpu.SemaphoreType.DMA,
                       pltpu.SemaphoreType.DMA],
    )
    def k(g_hbm, src_hbm, dst_hbm, gs_hbm, gd_hbm, idx_s, idx_d,
          buf_s, buf_d, sg_s, sg_d, ss_s, ss_d):
        wid = lax.axis_index("s") * NC + lax.axis_index("c")
        w0 = wid * per_w
        pltpu.sync_copy(src_hbm.at[wid], idx_s)
        pltpu.sync_copy(dst_hbm.at[wid], idx_d)

        def g_start(j, idxb, buf, sem):
            pltpu.make_async_copy(g_hbm.at[idxb.at[j, 0]], buf, sem).start()

        def g_wait(j, idxb, buf, sem):
            pltpu.make_async_copy(g_hbm.at[idxb.at[j, 0]], buf, sem).wait()

        def s_start(j, buf, out, sem):
            pltpu.make_async_copy(buf, out.at[pl.ds(w0 + j * CG, CG)],
                                  sem).start()

        def s_wait(j, buf, out, sem):
            pltpu.make_async_copy(buf, out.at[pl.ds(w0 + j * CG, CG)],
                                  sem).wait()

        g_start(0, idx_s, buf_s, sg_s)
        g_start(0, idx_d, buf_d, sg_d)

        @pl.loop(0, nch - 1)
        def _(j):
            g_wait(j, idx_s, buf_s, sg_s)
            s_start(j, buf_s, gs_hbm, ss_s)
            g_wait(j, idx_d, buf_d, sg_d)
            s_start(j, buf_d, gd_hbm, ss_d)
            s_wait(j, buf_s, gs_hbm, ss_s)
            g_start(j + 1, idx_s, buf_s, sg_s)
            s_wait(j, buf_d, gd_hbm, ss_d)
            g_start(j + 1, idx_d, buf_d, sg_d)

        g_wait(nch - 1, idx_s, buf_s, sg_s)
        s_start(nch - 1, buf_s, gs_hbm, ss_s)
        g_wait(nch - 1, idx_d, buf_d, sg_d)
        s_start(nch - 1, buf_d, gd_hbm, ss_d)
        s_wait(nch - 1, buf_s, gs_hbm, ss_s)
        s_wait(nch - 1, buf_d, gd_hbm, ss_d)

    return k(g, src4, dst4)


def _sc_scatter16(edge16, dst, zeros16):
    e, d = edge16.shape
    n = zeros16.shape[0]
    per_core = e // NC
    per_sub = per_core // NS
    nz = n // 10
    mesh = plsc.VectorSubcoreMesh(core_axis_name="c", subcore_axis_name="s",
                                  num_cores=NC, num_subcores=NS)

    @functools.partial(
        pl.kernel,
        out_type=jax.ShapeDtypeStruct((NC * n, d), jnp.float32),
        mesh=mesh,
        scratch_types=[pltpu.VMEM_SHARED((n, d), jnp.float32),
                       pltpu.VMEM((CS,), jnp.int32),
                       pltpu.VMEM((CS, d), jnp.float32),
                       pltpu.SemaphoreType.DMA],
    )
    def k(eo_hbm, dst_hbm, z_hbm, out_hbm, acc, idx_v, buf, sem):
        cid = lax.axis_index("c")
        sid = lax.axis_index("s")

        @pl.when(sid < 10)
        def _():
            pltpu.sync_copy(z_hbm.at[pl.ds(sid * nz, nz)],
                            acc.at[pl.ds(sid * nz, nz)])
        plsc.subcore_barrier()

        base0 = cid * per_core + sid * per_sub

        @pl.loop(0, per_sub, step=CS)
        def _(c):
            base = base0 + c
            pltpu.sync_copy(dst_hbm.at[pl.ds(base, CS)], idx_v)
            pltpu.sync_copy(eo_hbm.at[pl.ds(base, CS)], buf)
            pltpu.sync_copy(buf, acc.at[idx_v], add=True)
        plsc.subcore_barrier()

        @pl.when(sid < 10)
        def _():
            pltpu.sync_copy(acc.at[pl.ds(sid * nz, nz)],
                            out_hbm.at[pl.ds(cid * n + sid * nz, nz)])

    return k(edge16, dst, zeros16)


def _prep_body(an_ref, emb_ref, wmsg_ref, we_ref, g_ref, he_ref):
    an = an_ref[0]
    nelem = emb_ref.shape[0]
    oh = (jax.lax.broadcasted_iota(jnp.int32, (BN, nelem), 1) == an
          ).astype(jnp.float32)
    h = jnp.dot(oh, emb_ref[...], preferred_element_type=jnp.float32)
    g_ref[...] = jnp.dot(h, wmsg_ref[...], preferred_element_type=jnp.float32)
    he_ref[...] = jnp.dot(h, we_ref[...], preferred_element_type=jnp.float32)


def _edge_body(dv_ref, gs_ref, gd_ref, cen_ref, wrbf_ref, wf_ref, wupd_ref,
               we_ref, out_ref):
    dv = dv_ref[...]
    d2 = jnp.sum(dv * dv, axis=1, keepdims=True)
    d = jnp.sqrt(d2 + 1e-12)
    rbf = jnp.exp(-((d - cen_ref[...]) ** 2))
    t = jnp.dot(rbf, wrbf_ref[...], preferred_element_type=jnp.float32)
    a = gs_ref[...] + gd_ref[...] + t
    m = a * jax.lax.logistic(a)
    w = jnp.dot(wupd_ref[...], we_ref[...], preferred_element_type=jnp.float32)
    q = jnp.dot(m, w, preferred_element_type=jnp.float32)
    fscal = jnp.dot(m, wf_ref[...], preferred_element_type=jnp.float32)
    fe = fscal * (dv / d)
    out_ref[...] = jnp.concatenate(
        [q, fe, jnp.zeros((BE, 124), jnp.float32)], axis=1)


def _final_body(he_ref, p_ref, batch_ref, b1_ref, w2_ref, b2_ref,
                f_ref, e_ref, s_ref, cnt_ref):
    i = pl.program_id(0)
    nsteps = pl.num_programs(0)
    psum = p_ref[0] + p_ref[1]
    f_ref[...] = jnp.concatenate(
        [psum[:, 1:4], jnp.zeros((BN, 13), jnp.float32)], axis=1)
    e_node = he_ref[...] + psum[:, 0:1]
    batch_row = batch_ref[0]
    s_dim = e_ref.shape[0]
    oht = (jax.lax.broadcasted_iota(jnp.int32, (s_dim, BN), 0) == batch_row
           ).astype(jnp.float32)
    e_part = jnp.dot(oht, e_node, preferred_element_type=jnp.float32)
    c_part = jnp.sum(oht, axis=1, keepdims=True)

    @pl.when(i == 0)
    def _():
        e_ref[...] = jnp.zeros_like(e_ref)
        cnt_ref[...] = jnp.zeros_like(cnt_ref)

    e_ref[...] += e_part
    cnt_ref[...] += c_part

    @pl.when(i == nsteps - 1)
    def _():
        srow = jnp.dot(jnp.tanh(b1_ref[...]), w2_ref[...],
                       preferred_element_type=jnp.float32) + b2_ref[...]
        s_ref[...] = cnt_ref[...] * srow


def _prep(atomic_numbers, atom_emb, W_msg, W_energy):
    n = atomic_numbers.shape[0]
    nelem, d = atom_emb.shape
    an3 = atomic_numbers.reshape(n // BN, BN, 1)
    return pl.pallas_call(
        _prep_body,
        grid=(n // BN,),
        in_specs=[
            pl.BlockSpec((1, BN, 1), lambda i: (i, 0, 0)),
            pl.BlockSpec((nelem, d), lambda i: (0, 0)),
            pl.BlockSpec((d, d), lambda i: (0, 0)),
            pl.BlockSpec((d, 1), lambda i: (0, 0)),
        ],
        out_specs=[
            pl.BlockSpec((BN, d), lambda i: (i, 0)),
            pl.BlockSpec((BN, 1), lambda i: (i, 0)),
        ],
        out_shape=[
            jax.ShapeDtypeStruct((n, d), jnp.float32),
            jax.ShapeDtypeStruct((n, 1), jnp.float32),
        ],
    )(an3, atom_emb, W_msg, W_energy)


def _edge(distance_vec, g_src, g_dst, centers, W_rbf, W_force, W_upd,
          W_energy):
    e, d = g_src.shape
    nrbf = W_rbf.shape[0]
    cen2 = centers.reshape(1, nrbf)
    return pl.pallas_call(
        _edge_body,
        grid=(e // BE,),
        in_specs=[
            pl.BlockSpec((BE, 3), lambda i: (i, 0)),
            pl.BlockSpec((BE, d), lambda i: (i, 0)),
            pl.BlockSpec((BE, d), lambda i: (i, 0)),
            pl.BlockSpec((1, nrbf), lambda i: (0, 0)),
            pl.BlockSpec((nrbf, d), lambda i: (0, 0)),
            pl.BlockSpec((d, 1), lambda i: (0, 0)),
            pl.BlockSpec((d, d), lambda i: (0, 0)),
            pl.BlockSpec((d, 1), lambda i: (0, 0)),
        ],
        out_specs=pl.BlockSpec((BE, 128), lambda i: (i, 0)),
        out_shape=jax.ShapeDtypeStruct((e, 128), jnp.float32),
    )(distance_vec, g_src, g_dst, cen2, W_rbf, W_force, W_upd, W_energy)


def _final(he, partials, batch, b1, W2, b2, num_struct):
    n = he.shape[0]
    d = b1.shape[0]
    batch3 = batch.reshape(n // BN, 1, BN)
    return pl.pallas_call(
        _final_body,
        grid=(n // BN,),
        in_specs=[
            pl.BlockSpec((BN, 1), lambda i: (i, 0)),
            pl.BlockSpec((2, BN, 128), lambda i: (0, i, 0)),
            pl.BlockSpec((1, 1, BN), lambda i: (i, 0, 0)),
            pl.BlockSpec((1, d), lambda i: (0, 0)),
            pl.BlockSpec((d, 6), lambda i: (0, 0)),
            pl.BlockSpec((1, 6), lambda i: (0, 0)),
        ],
        out_specs=[
            pl.BlockSpec((BN, 16), lambda i: (i, 0)),
            pl.BlockSpec((num_struct, 1), lambda i: (0, 0)),
            pl.BlockSpec((num_struct, 6), lambda i: (0, 0)),
        ],
        out_shape=[
            jax.ShapeDtypeStruct((n, 16), jnp.float32),
            jax.ShapeDtypeStruct((num_struct, 1), jnp.float32),
            jax.ShapeDtypeStruct((num_struct, 6), jnp.float32),
        ],
        scratch_shapes=[pltpu.VMEM((num_struct, 1), jnp.float32)],
    )(he, partials, batch3, b1.reshape(1, d), W2, b2.reshape(1, 6))


def kernel(pos, atomic_numbers, edge_index, distance_vec, batch, atom_emb,
           centers, W_msg, W_rbf, W_upd, W_energy, W_force, W1, b1, W2, b2):
    n = pos.shape[0]
    src = edge_index[0]
    dst = edge_index[1]
    g, he = _prep(atomic_numbers, atom_emb, W_msg, W_energy)
    g_src, g_dst = _sc_gather(g, src, dst)
    edge16 = _edge(distance_vec, g_src, g_dst, centers, W_rbf, W_force,
                   W_upd, W_energy)
    zeros_acc = jnp.zeros((n, 128), jnp.float32)
    p2 = _sc_scatter16(edge16, dst, zeros_acc)
    partials = p2.reshape(2, n, 128)
    f16, e_out, stress = _final(he, partials, batch, b1, W2, b2, 64)
    forces = f16[:, :3]
    energy = e_out[:, 0]
    return forces, energy, stress


# speedup vs baseline: 3.6862x; 1.0148x over previous
import functools
import jax
import jax.numpy as jnp
from jax import lax
from jax.experimental import pallas as pl
from jax.experimental.pallas import tpu as pltpu
from jax.experimental.pallas import tpu_sc as plsc

BN = 1000
BE = 1600
NC, NS = 2, 16
NW = NC * NS
CG = 80
CS = 80


def _sc_gather(g, src, dst):
    e = src.shape[0]
    n, d = g.shape
    per_w = e // NW
    nch = per_w // CG
    mesh = plsc.VectorSubcoreMesh(core_axis_name="c", subcore_axis_name="s",
                                  num_cores=NC, num_subcores=NS)

    @functools.partial(
        pl.kernel,
        out_type=[jax.ShapeDtypeStruct((e, d), jnp.float32),
                  jax.ShapeDtypeStruct((e, d), jnp.float32)],
        mesh=mesh,
        scratch_types=[pltpu.VMEM((CG,), jnp.int32),
                       pltpu.VMEM((CG,), jnp.int32),
                       pltpu.VMEM((CG, d), jnp.float32),
                       pltpu.VMEM((CG, d), jnp.float32),
                       pltpu.SemaphoreType.DMA,
                       pltpu.SemaphoreType.DMA,
                       pltpu.SemaphoreType.DMA,
                       pltpu.SemaphoreType.DMA,
                       pltpu.SemaphoreType.DMA,
                       pltpu.SemaphoreType.DMA],
    )
    def k(g_hbm, src_hbm, dst_hbm, gs_hbm, gd_hbm, idx_s, idx_d,
          buf_s, buf_d, sg_s, sg_d, ss_s, ss_d, si_s, si_d):
        wid = lax.axis_index("s") * NC + lax.axis_index("c")
        w0 = wid * per_w

        def il_start(j, src1, idxb, sem):
            pltpu.make_async_copy(src1.at[pl.ds(w0 + j * CG, CG)], idxb,
                                  sem).start()

        def il_wait(j, src1, idxb, sem):
            pltpu.make_async_copy(src1.at[pl.ds(w0 + j * CG, CG)], idxb,
                                  sem).wait()

        def g_start(idxb, buf, sem):
            pltpu.make_async_copy(g_hbm.at[idxb], buf, sem).start()

        def g_wait(idxb, buf, sem):
            pltpu.make_async_copy(g_hbm.at[idxb], buf, sem).wait()

        def s_start(j, buf, out, sem):
            pltpu.make_async_copy(buf, out.at[pl.ds(w0 + j * CG, CG)],
                                  sem).start()

        def s_wait(j, buf, out, sem):
            pltpu.make_async_copy(buf, out.at[pl.ds(w0 + j * CG, CG)],
                                  sem).wait()

        il_start(0, src_hbm, idx_s, si_s)
        il_start(0, dst_hbm, idx_d, si_d)
        il_wait(0, src_hbm, idx_s, si_s)
        il_wait(0, dst_hbm, idx_d, si_d)
        g_start(idx_s, buf_s, sg_s)
        g_start(idx_d, buf_d, sg_d)

        @pl.loop(0, nch - 1)
        def _(j):
            g_wait(idx_s, buf_s, sg_s)
            il_start(j + 1, src_hbm, idx_s, si_s)
            s_start(j, buf_s, gs_hbm, ss_s)
            g_wait(idx_d, buf_d, sg_d)
            il_start(j + 1, dst_hbm, idx_d, si_d)
            s_start(j, buf_d, gd_hbm, ss_d)
            s_wait(j, buf_s, gs_hbm, ss_s)
            il_wait(j + 1, src_hbm, idx_s, si_s)
            g_start(idx_s, buf_s, sg_s)
            s_wait(j, buf_d, gd_hbm, ss_d)
            il_wait(j + 1, dst_hbm, idx_d, si_d)
            g_start(idx_d, buf_d, sg_d)

        g_wait(idx_s, buf_s, sg_s)
        s_start(nch - 1, buf_s, gs_hbm, ss_s)
        g_wait(idx_d, buf_d, sg_d)
        s_start(nch - 1, buf_d, gd_hbm, ss_d)
        s_wait(nch - 1, buf_s, gs_hbm, ss_s)
        s_wait(nch - 1, buf_d, gd_hbm, ss_d)

    return k(g, src, dst)


def _sc_scatter16(edge16, dst, zeros16):
    e, d = edge16.shape
    n = zeros16.shape[0]
    per_core = e // NC
    per_sub = per_core // NS
    nz = n // 10
    mesh = plsc.VectorSubcoreMesh(core_axis_name="c", subcore_axis_name="s",
                                  num_cores=NC, num_subcores=NS)

    nchs = per_sub // CS
    half = nchs // 2

    @functools.partial(
        pl.kernel,
        out_type=jax.ShapeDtypeStruct((NC * n, d), jnp.float32),
        mesh=mesh,
        scratch_types=[pltpu.VMEM_SHARED((n, d), jnp.float32),
                       pltpu.VMEM((CS,), jnp.int32),
                       pltpu.VMEM((CS,), jnp.int32),
                       pltpu.VMEM((CS, d), jnp.float32),
                       pltpu.VMEM((CS, d), jnp.float32),
                       pltpu.SemaphoreType.DMA,
                       pltpu.SemaphoreType.DMA,
                       pltpu.SemaphoreType.DMA,
                       pltpu.SemaphoreType.DMA,
                       pltpu.SemaphoreType.DMA,
                       pltpu.SemaphoreType.DMA],
    )
    def k(eo_hbm, dst_hbm, z_hbm, out_hbm, acc, idx0, idx1, buf0, buf1,
          si0, si1, sp0, sp1, sa0, sa1):
        cid = lax.axis_index("c")
        sid = lax.axis_index("s")

        @pl.when(sid < 10)
        def _():
            pltpu.sync_copy(z_hbm.at[pl.ds(sid * nz, nz)],
                            acc.at[pl.ds(sid * nz, nz)])
        plsc.subcore_barrier()

        base0 = cid * per_core + sid * per_sub

        def il(j, idxb, sem):
            return pltpu.make_async_copy(
                dst_hbm.at[pl.ds(base0 + j * CS, CS)], idxb, sem)

        def pld(j, bufb, sem):
            return pltpu.make_async_copy(
                eo_hbm.at[pl.ds(base0 + j * CS, CS)], bufb, sem)

        def ad(bufb, idxb, sem):
            return pltpu.async_copy(bufb, acc.at[idxb], sem, add=True)

        @pl.loop(0, nchs)
        def _(j):
            pltpu.sync_copy(dst_hbm.at[pl.ds(base0 + j * CS, CS)], idx0)
            pltpu.sync_copy(eo_hbm.at[pl.ds(base0 + j * CS, CS)], buf0)
            pltpu.sync_copy(buf0, acc.at[idx0], add=True)
        plsc.subcore_barrier()

        @pl.when(sid < 10)
        def _():
            pltpu.sync_copy(acc.at[pl.ds(sid * nz, nz)],
                            out_hbm.at[pl.ds(cid * n + sid * nz, nz)])

    return k(edge16, dst, zeros16)


def _prep_body(an_ref, emb_ref, wmsg_ref, we_ref, g_ref, he_ref):
    an = an_ref[0]
    nelem = emb_ref.shape[0]
    oh = (jax.lax.broadcasted_iota(jnp.int32, (BN, nelem), 1) == an
          ).astype(jnp.float32)
    h = jnp.dot(oh, emb_ref[...], preferred_element_type=jnp.float32)
    g_ref[...] = jnp.dot(h, wmsg_ref[...], preferred_element_type=jnp.float32)
    he_ref[...] = jnp.dot(h, we_ref[...], preferred_element_type=jnp.float32)


def _edge_body(dv_ref, gs_ref, gd_ref, cen_ref, wrbf_ref, wf_ref, wupd_ref,
               we_ref, out_ref):
    dv = dv_ref[...]
    d2 = jnp.sum(dv * dv, axis=1, keepdims=True)
    d = jnp.sqrt(d2 + 1e-12)
    rbf = jnp.exp(-((d - cen_ref[...]) ** 2))
    t = jnp.dot(rbf, wrbf_ref[...], preferred_element_type=jnp.float32)
    a = gs_ref[...] + gd_ref[...] + t
    m = a * jax.lax.logistic(a)
    w = jnp.dot(wupd_ref[...], we_ref[...], preferred_element_type=jnp.float32)
    q = jnp.dot(m, w, preferred_element_type=jnp.float32)
    fscal = jnp.dot(m, wf_ref[...], preferred_element_type=jnp.float32)
    fe = fscal * (dv / d)
    out_ref[...] = jnp.concatenate(
        [q, fe, jnp.zeros((BE, 124), jnp.float32)], axis=1)


def _final_body(he_ref, p_ref, batch_ref, b1_ref, w2_ref, b2_ref,
                f_ref, e_ref, s_ref, cnt_ref):
    i = pl.program_id(0)
    nsteps = pl.num_programs(0)
    psum = p_ref[0] + p_ref[1]
    f_ref[...] = jnp.concatenate(
        [psum[:, 1:4], jnp.zeros((BN, 13), jnp.float32)], axis=1)
    e_node = he_ref[...] + psum[:, 0:1]
    batch_row = batch_ref[0]
    s_dim = e_ref.shape[0]
    oht = (jax.lax.broadcasted_iota(jnp.int32, (s_dim, BN), 0) == batch_row
           ).astype(jnp.float32)
    e_part = jnp.dot(oht, e_node, preferred_element_type=jnp.float32)
    c_part = jnp.sum(oht, axis=1, keepdims=True)

    @pl.when(i == 0)
    def _():
        e_ref[...] = jnp.zeros_like(e_ref)
        cnt_ref[...] = jnp.zeros_like(cnt_ref)

    e_ref[...] += e_part
    cnt_ref[...] += c_part

    @pl.when(i == nsteps - 1)
    def _():
        srow = jnp.dot(jnp.tanh(b1_ref[...]), w2_ref[...],
                       preferred_element_type=jnp.float32) + b2_ref[...]
        s_ref[...] = cnt_ref[...] * srow


def _prep(atomic_numbers, atom_emb, W_msg, W_energy):
    n = atomic_numbers.shape[0]
    nelem, d = atom_emb.shape
    an3 = atomic_numbers.reshape(n // BN, BN, 1)
    return pl.pallas_call(
        _prep_body,
        grid=(n // BN,),
        in_specs=[
            pl.BlockSpec((1, BN, 1), lambda i: (i, 0, 0)),
            pl.BlockSpec((nelem, d), lambda i: (0, 0)),
            pl.BlockSpec((d, d), lambda i: (0, 0)),
            pl.BlockSpec((d, 1), lambda i: (0, 0)),
        ],
        out_specs=[
            pl.BlockSpec((BN, d), lambda i: (i, 0)),
            pl.BlockSpec((BN, 1), lambda i: (i, 0)),
        ],
        out_shape=[
            jax.ShapeDtypeStruct((n, d), jnp.float32),
            jax.ShapeDtypeStruct((n, 1), jnp.float32),
        ],
    )(an3, atom_emb, W_msg, W_energy)


def _edge(distance_vec, g_src, g_dst, centers, W_rbf, W_force, W_upd,
          W_energy):
    e, d = g_src.shape
    nrbf = W_rbf.shape[0]
    cen2 = centers.reshape(1, nrbf)
    return pl.pallas_call(
        _edge_body,
        grid=(e // BE,),
        in_specs=[
            pl.BlockSpec((BE, 3), lambda i: (i, 0)),
            pl.BlockSpec((BE, d), lambda i: (i, 0)),
            pl.BlockSpec((BE, d), lambda i: (i, 0)),
            pl.BlockSpec((1, nrbf), lambda i: (0, 0)),
            pl.BlockSpec((nrbf, d), lambda i: (0, 0)),
            pl.BlockSpec((d, 1), lambda i: (0, 0)),
            pl.BlockSpec((d, d), lambda i: (0, 0)),
            pl.BlockSpec((d, 1), lambda i: (0, 0)),
        ],
        out_specs=pl.BlockSpec((BE, 128), lambda i: (i, 0)),
        out_shape=jax.ShapeDtypeStruct((e, 128), jnp.float32),
    )(distance_vec, g_src, g_dst, cen2, W_rbf, W_force, W_upd, W_energy)


def _final(he, partials, batch, b1, W2, b2, num_struct):
    n = he.shape[0]
    d = b1.shape[0]
    batch3 = batch.reshape(n // BN, 1, BN)
    return pl.pallas_call(
        _final_body,
        grid=(n // BN,),
        in_specs=[
            pl.BlockSpec((BN, 1), lambda i: (i, 0)),
            pl.BlockSpec((2, BN, 128), lambda i: (0, i, 0)),
            pl.BlockSpec((1, 1, BN), lambda i: (i, 0, 0)),
            pl.BlockSpec((1, d), lambda i: (0, 0)),
            pl.BlockSpec((d, 6), lambda i: (0, 0)),
            pl.BlockSpec((1, 6), lambda i: (0, 0)),
        ],
        out_specs=[
            pl.BlockSpec((BN, 16), lambda i: (i, 0)),
            pl.BlockSpec((num_struct, 1), lambda i: (0, 0)),
            pl.BlockSpec((num_struct, 6), lambda i: (0, 0)),
        ],
        out_shape=[
            jax.ShapeDtypeStruct((n, 16), jnp.float32),
            jax.ShapeDtypeStruct((num_struct, 1), jnp.float32),
            jax.ShapeDtypeStruct((num_struct, 6), jnp.float32),
        ],
        scratch_shapes=[pltpu.VMEM((num_struct, 1), jnp.float32)],
    )(he, partials, batch3, b1.reshape(1, d), W2, b2.reshape(1, 6))


def kernel(pos, atomic_numbers, edge_index, distance_vec, batch, atom_emb,
           centers, W_msg, W_rbf, W_upd, W_energy, W_force, W1, b1, W2, b2):
    n = pos.shape[0]
    src = edge_index[0]
    dst = edge_index[1]
    g, he = _prep(atomic_numbers, atom_emb, W_msg, W_energy)
    g_src, g_dst = _sc_gather(g, src, dst)
    edge16 = _edge(distance_vec, g_src, g_dst, centers, W_rbf, W_force,
                   W_upd, W_energy)
    zeros_acc = jnp.zeros((n, 128), jnp.float32)
    p2 = _sc_scatter16(edge16, dst, zeros_acc)
    partials = p2.reshape(2, n, 128)
    f16, e_out, stress = _final(he, partials, batch, b1, W2, b2, 64)
    forces = f16[:, :3]
    energy = e_out[:, 0]
    return forces, energy, stress


# speedup vs baseline: 4.1305x; 1.1206x over previous
import functools
import jax
import jax.numpy as jnp
from jax import lax
from jax.experimental import pallas as pl
from jax.experimental.pallas import tpu as pltpu
from jax.experimental.pallas import tpu_sc as plsc

BN = 1000
BE = 1600
NC, NS = 2, 16
NW = NC * NS
CG = 80
CS = 80


def _sc_gather(g, src, dst):
    e = src.shape[0]
    n, d = g.shape
    per_w = e // NW
    nch = per_w // CG
    mesh = plsc.VectorSubcoreMesh(core_axis_name="c", subcore_axis_name="s",
                                  num_cores=NC, num_subcores=NS)

    @functools.partial(
        pl.kernel,
        out_type=[jax.ShapeDtypeStruct((e, d), jnp.float32),
                  jax.ShapeDtypeStruct((e, d), jnp.float32)],
        mesh=mesh,
        scratch_types=[pltpu.VMEM((CG,), jnp.int32),
                       pltpu.VMEM((CG,), jnp.int32),
                       pltpu.VMEM((CG, d), jnp.float32),
                       pltpu.VMEM((CG, d), jnp.float32),
                       pltpu.SemaphoreType.DMA,
                       pltpu.SemaphoreType.DMA,
                       pltpu.SemaphoreType.DMA,
                       pltpu.SemaphoreType.DMA,
                       pltpu.SemaphoreType.DMA,
                       pltpu.SemaphoreType.DMA],
    )
    def k(g_hbm, src_hbm, dst_hbm, gs_hbm, gd_hbm, idx_s, idx_d,
          buf_s, buf_d, sg_s, sg_d, ss_s, ss_d, si_s, si_d):
        wid = lax.axis_index("s") * NC + lax.axis_index("c")
        w0 = wid * per_w

        def il_start(j, src1, idxb, sem):
            pltpu.make_async_copy(src1.at[pl.ds(w0 + j * CG, CG)], idxb,
                                  sem).start()

        def il_wait(j, src1, idxb, sem):
            pltpu.make_async_copy(src1.at[pl.ds(w0 + j * CG, CG)], idxb,
                                  sem).wait()

        def g_start(idxb, buf, sem):
            pltpu.make_async_copy(g_hbm.at[idxb], buf, sem).start()

        def g_wait(idxb, buf, sem):
            pltpu.make_async_copy(g_hbm.at[idxb], buf, sem).wait()

        def s_start(j, buf, out, sem):
            pltpu.make_async_copy(buf, out.at[pl.ds(w0 + j * CG, CG)],
                                  sem).start()

        def s_wait(j, buf, out, sem):
            pltpu.make_async_copy(buf, out.at[pl.ds(w0 + j * CG, CG)],
                                  sem).wait()

        il_start(0, src_hbm, idx_s, si_s)
        il_start(0, dst_hbm, idx_d, si_d)
        il_wait(0, src_hbm, idx_s, si_s)
        il_wait(0, dst_hbm, idx_d, si_d)
        g_start(idx_s, buf_s, sg_s)
        g_start(idx_d, buf_d, sg_d)

        @pl.loop(0, nch - 1)
        def _(j):
            g_wait(idx_s, buf_s, sg_s)
            il_start(j + 1, src_hbm, idx_s, si_s)
            s_start(j, buf_s, gs_hbm, ss_s)
            g_wait(idx_d, buf_d, sg_d)
            il_start(j + 1, dst_hbm, idx_d, si_d)
            s_start(j, buf_d, gd_hbm, ss_d)
            s_wait(j, buf_s, gs_hbm, ss_s)
            il_wait(j + 1, src_hbm, idx_s, si_s)
            g_start(idx_s, buf_s, sg_s)
            s_wait(j, buf_d, gd_hbm, ss_d)
            il_wait(j + 1, dst_hbm, idx_d, si_d)
            g_start(idx_d, buf_d, sg_d)

        g_wait(idx_s, buf_s, sg_s)
        s_start(nch - 1, buf_s, gs_hbm, ss_s)
        g_wait(idx_d, buf_d, sg_d)
        s_start(nch - 1, buf_d, gd_hbm, ss_d)
        s_wait(nch - 1, buf_s, gs_hbm, ss_s)
        s_wait(nch - 1, buf_d, gd_hbm, ss_d)

    return k(g, src, dst)


def _sc_scatter16(edge16, dst, zeros16):
    e, d = edge16.shape
    n = zeros16.shape[0]
    per_core = e // NC
    per_sub = per_core // NS
    nz = n // 10
    mesh = plsc.VectorSubcoreMesh(core_axis_name="c", subcore_axis_name="s",
                                  num_cores=NC, num_subcores=NS)

    nchs = per_sub // CS
    half = nchs // 2

    @functools.partial(
        pl.kernel,
        out_type=jax.ShapeDtypeStruct((NC * n, d), jnp.float32),
        mesh=mesh,
        scratch_types=[pltpu.VMEM_SHARED((n, d), jnp.float32),
                       pltpu.VMEM((CS,), jnp.int32),
                       pltpu.VMEM((CS,), jnp.int32),
                       pltpu.VMEM((CS, d), jnp.float32),
                       pltpu.VMEM((CS, d), jnp.float32),
                       pltpu.SemaphoreType.DMA,
                       pltpu.SemaphoreType.DMA,
                       pltpu.SemaphoreType.DMA,
                       pltpu.SemaphoreType.DMA,
                       pltpu.SemaphoreType.DMA,
                       pltpu.SemaphoreType.DMA],
    )
    def k(eo_hbm, dst_hbm, z_hbm, out_hbm, acc, idx0, idx1, buf0, buf1,
          si0, si1, sp0, sp1, sa0, sa1):
        cid = lax.axis_index("c")
        sid = lax.axis_index("s")

        @pl.when(sid < 10)
        def _():
            pltpu.sync_copy(z_hbm.at[pl.ds(sid * nz, nz)],
                            acc.at[pl.ds(sid * nz, nz)])
        plsc.subcore_barrier()

        base0 = cid * per_core + sid * per_sub

        def il(j, idxb, sem):
            return pltpu.make_async_copy(
                dst_hbm.at[pl.ds(base0 + j * CS, CS)], idxb, sem)

        def pld(j, bufb, sem):
            return pltpu.make_async_copy(
                eo_hbm.at[pl.ds(base0 + j * CS, CS)], bufb, sem)

        def ad(bufb, idxb, sem):
            return pltpu.async_copy(bufb, acc.at[idxb], sem, add=True)

        il(0, idx0, si0).start()
        pld(0, buf0, sp0).start()

        @pl.loop(0, half)
        def _(t):
            j0 = 2 * t
            j1 = 2 * t + 1
            il(j0, idx0, si0).wait()
            pld(j0, buf0, sp0).wait()
            a0 = ad(buf0, idx0, sa0)
            il(j1, idx1, si1).start()
            pld(j1, buf1, sp1).start()
            a0.wait()
            il(j1, idx1, si1).wait()
            pld(j1, buf1, sp1).wait()
            a1 = ad(buf1, idx1, sa1)
            il(j0 + 2, idx0, si0).start()
            pld(j0 + 2, buf0, sp0).start()
            a1.wait()

        il(nchs - 1, idx0, si0).wait()
        pld(nchs - 1, buf0, sp0).wait()
        a_last = ad(buf0, idx0, sa0)
        a_last.wait()
        plsc.subcore_barrier()

        @pl.when(sid < 10)
        def _():
            pltpu.sync_copy(acc.at[pl.ds(sid * nz, nz)],
                            out_hbm.at[pl.ds(cid * n + sid * nz, nz)])

    return k(edge16, dst, zeros16)


def _prep_body(an_ref, emb_ref, wmsg_ref, we_ref, g_ref, he_ref):
    an = an_ref[0]
    nelem = emb_ref.shape[0]
    oh = (jax.lax.broadcasted_iota(jnp.int32, (BN, nelem), 1) == an
          ).astype(jnp.float32)
    h = jnp.dot(oh, emb_ref[...], preferred_element_type=jnp.float32)
    g_ref[...] = jnp.dot(h, wmsg_ref[...], preferred_element_type=jnp.float32)
    he_ref[...] = jnp.dot(h, we_ref[...], preferred_element_type=jnp.float32)


def _edge_body(dv_ref, gs_ref, gd_ref, cen_ref, wrbf_ref, wf_ref, wupd_ref,
               we_ref, out_ref):
    dv = dv_ref[...]
    d2 = jnp.sum(dv * dv, axis=1, keepdims=True)
    d = jnp.sqrt(d2 + 1e-12)
    rbf = jnp.exp(-((d - cen_ref[...]) ** 2))
    t = jnp.dot(rbf, wrbf_ref[...], preferred_element_type=jnp.float32)
    a = gs_ref[...] + gd_ref[...] + t
    m = a * jax.lax.logistic(a)
    w = jnp.dot(wupd_ref[...], we_ref[...], preferred_element_type=jnp.float32)
    q = jnp.dot(m, w, preferred_element_type=jnp.float32)
    fscal = jnp.dot(m, wf_ref[...], preferred_element_type=jnp.float32)
    fe = fscal * (dv / d)
    out_ref[...] = jnp.concatenate(
        [q, fe, jnp.zeros((BE, 124), jnp.float32)], axis=1)


def _final_body(he_ref, p_ref, batch_ref, b1_ref, w2_ref, b2_ref,
                f_ref, e_ref, s_ref, cnt_ref):
    i = pl.program_id(0)
    nsteps = pl.num_programs(0)
    psum = p_ref[0] + p_ref[1]
    f_ref[...] = jnp.concatenate(
        [psum[:, 1:4], jnp.zeros((BN, 13), jnp.float32)], axis=1)
    e_node = he_ref[...] + psum[:, 0:1]
    batch_row = batch_ref[0]
    s_dim = e_ref.shape[0]
    oht = (jax.lax.broadcasted_iota(jnp.int32, (s_dim, BN), 0) == batch_row
           ).astype(jnp.float32)
    e_part = jnp.dot(oht, e_node, preferred_element_type=jnp.float32)
    c_part = jnp.sum(oht, axis=1, keepdims=True)

    @pl.when(i == 0)
    def _():
        e_ref[...] = jnp.zeros_like(e_ref)
        cnt_ref[...] = jnp.zeros_like(cnt_ref)

    e_ref[...] += e_part
    cnt_ref[...] += c_part

    @pl.when(i == nsteps - 1)
    def _():
        srow = jnp.dot(jnp.tanh(b1_ref[...]), w2_ref[...],
                       preferred_element_type=jnp.float32) + b2_ref[...]
        s_ref[...] = cnt_ref[...] * srow


def _prep(atomic_numbers, atom_emb, W_msg, W_energy):
    n = atomic_numbers.shape[0]
    nelem, d = atom_emb.shape
    an3 = atomic_numbers.reshape(n // BN, BN, 1)
    return pl.pallas_call(
        _prep_body,
        grid=(n // BN,),
        in_specs=[
            pl.BlockSpec((1, BN, 1), lambda i: (i, 0, 0)),
            pl.BlockSpec((nelem, d), lambda i: (0, 0)),
            pl.BlockSpec((d, d), lambda i: (0, 0)),
            pl.BlockSpec((d, 1), lambda i: (0, 0)),
        ],
        out_specs=[
            pl.BlockSpec((BN, d), lambda i: (i, 0)),
            pl.BlockSpec((BN, 1), lambda i: (i, 0)),
        ],
        out_shape=[
            jax.ShapeDtypeStruct((n, d), jnp.float32),
            jax.ShapeDtypeStruct((n, 1), jnp.float32),
        ],
    )(an3, atom_emb, W_msg, W_energy)


def _edge(distance_vec, g_src, g_dst, centers, W_rbf, W_force, W_upd,
          W_energy):
    e, d = g_src.shape
    nrbf = W_rbf.shape[0]
    cen2 = centers.reshape(1, nrbf)
    return pl.pallas_call(
        _edge_body,
        grid=(e // BE,),
        in_specs=[
            pl.BlockSpec((BE, 3), lambda i: (i, 0)),
            pl.BlockSpec((BE, d), lambda i: (i, 0)),
            pl.BlockSpec((BE, d), lambda i: (i, 0)),
            pl.BlockSpec((1, nrbf), lambda i: (0, 0)),
            pl.BlockSpec((nrbf, d), lambda i: (0, 0)),
            pl.BlockSpec((d, 1), lambda i: (0, 0)),
            pl.BlockSpec((d, d), lambda i: (0, 0)),
            pl.BlockSpec((d, 1), lambda i: (0, 0)),
        ],
        out_specs=pl.BlockSpec((BE, 128), lambda i: (i, 0)),
        out_shape=jax.ShapeDtypeStruct((e, 128), jnp.float32),
    )(distance_vec, g_src, g_dst, cen2, W_rbf, W_force, W_upd, W_energy)


def _final(he, partials, batch, b1, W2, b2, num_struct):
    n = he.shape[0]
    d = b1.shape[0]
    batch3 = batch.reshape(n // BN, 1, BN)
    return pl.pallas_call(
        _final_body,
        grid=(n // BN,),
        in_specs=[
            pl.BlockSpec((BN, 1), lambda i: (i, 0)),
            pl.BlockSpec((2, BN, 128), lambda i: (0, i, 0)),
            pl.BlockSpec((1, 1, BN), lambda i: (i, 0, 0)),
            pl.BlockSpec((1, d), lambda i: (0, 0)),
            pl.BlockSpec((d, 6), lambda i: (0, 0)),
            pl.BlockSpec((1, 6), lambda i: (0, 0)),
        ],
        out_specs=[
            pl.BlockSpec((BN, 16), lambda i: (i, 0)),
            pl.BlockSpec((num_struct, 1), lambda i: (0, 0)),
            pl.BlockSpec((num_struct, 6), lambda i: (0, 0)),
        ],
        out_shape=[
            jax.ShapeDtypeStruct((n, 16), jnp.float32),
            jax.ShapeDtypeStruct((num_struct, 1), jnp.float32),
            jax.ShapeDtypeStruct((num_struct, 6), jnp.float32),
        ],
        scratch_shapes=[pltpu.VMEM((num_struct, 1), jnp.float32)],
    )(he, partials, batch3, b1.reshape(1, d), W2, b2.reshape(1, 6))


def kernel(pos, atomic_numbers, edge_index, distance_vec, batch, atom_emb,
           centers, W_msg, W_rbf, W_upd, W_energy, W_force, W1, b1, W2, b2):
    n = pos.shape[0]
    src = edge_index[0]
    dst = edge_index[1]
    g, he = _prep(atomic_numbers, atom_emb, W_msg, W_energy)
    g_src, g_dst = _sc_gather(g, src, dst)
    edge16 = _edge(distance_vec, g_src, g_dst, centers, W_rbf, W_force,
                   W_upd, W_energy)
    zeros_acc = jnp.zeros((n, 128), jnp.float32)
    p2 = _sc_scatter16(edge16, dst, zeros_acc)
    partials = p2.reshape(2, n, 128)
    f16, e_out, stress = _final(he, partials, batch, b1, W2, b2, 64)
    forces = f16[:, :3]
    energy = e_out[:, 0]
    return forces, energy, stress
